# baseline (speedup 1.0000x reference)
import jax
import jax.numpy as jnp
from jax.experimental import pallas as pl
from jax.experimental.pallas import tpu as pltpu

N = 4096
REL = 2
NB = 2
H0 = 64
H1 = 64
BLK = 128
NBLK = N // BLK
CACHE_BLKS = 0


def _bf(x):
    return x.astype(jnp.bfloat16).astype(jnp.float32)


def _rgcn_body(c0_ref, c1_ref, adj_ref, bw0_ref, bw1_ref,
               out1_ref, fsum_ref, out0_scr, w0_scr, w1_scr, h_scr, cache_scr):
    p = pl.program_id(0)
    i = pl.program_id(1)

    @pl.when(jnp.logical_and(p == 0, i == 0))
    def _init():
        fsum_ref[...] = jnp.zeros_like(fsum_ref)
        for r in range(REL):
            w0_scr[r] = (_bf(c0_ref[r, 0]) * _bf(bw0_ref[0])
                         + _bf(c0_ref[r, 1]) * _bf(bw0_ref[1])
                         ).astype(jnp.bfloat16)
            w1_scr[r] = (_bf(c1_ref[r, 0]) * _bf(bw1_ref[0])
                         + _bf(c1_ref[r, 1]) * _bf(bw1_ref[1])
                         ).astype(jnp.bfloat16)

    @pl.when(p == 0)
    def _phase0():
        acc = jnp.zeros((BLK, H0), dtype=jnp.float32)
        for r in range(REL):
            ab = adj_ref[r].astype(jnp.bfloat16)

            @pl.when(i < CACHE_BLKS)
            def _park():
                cache_scr[r, pl.ds(i * BLK, BLK), :] = ab

            acc = acc + jnp.dot(ab, w0_scr[r],
                                preferred_element_type=jnp.float32)
        out0_scr[pl.ds(i * BLK, BLK), :] = acc
        fsum_ref[0:1, :] = fsum_ref[0:1, :] + jnp.sum(acc, axis=0, keepdims=True)

    @pl.when(jnp.logical_and(p == 1, i == 0))
    def _mk_h():
        h_scr[...] = jnp.maximum(out0_scr[...], 0.0).astype(jnp.bfloat16)

    @pl.when(p == 1)
    def _phase1():
        j = NBLK - 1 - i
        h = h_scr[...]

        def emit(srcs):
            acc = jnp.zeros((BLK, H1), dtype=jnp.float32)
            for r in range(REL):
                t = jnp.dot(srcs[r], h, preferred_element_type=jnp.float32)
                acc = acc + jnp.dot(t.astype(jnp.bfloat16), w1_scr[r],
                                    preferred_element_type=jnp.float32)
            out1_ref[...] = acc
            fsum_ref[1:2, :] = (fsum_ref[1:2, :]
                                + jnp.sum(acc, axis=0, keepdims=True))

        @pl.when(j >= CACHE_BLKS)
        def _from_hbm():
            emit([adj_ref[r].astype(jnp.bfloat16) for r in range(REL)])

        @pl.when(j < CACHE_BLKS)
        def _from_cache():
            emit([cache_scr[r, pl.ds(j * BLK, BLK), :] for r in range(REL)])


def _adj_index(p, i):
    return (0, jnp.where(p == 0, i, jnp.maximum(NBLK - 1 - i, CACHE_BLKS)), 0)


def kernel(adj, basis_weight0, basis_coeff0, basis_weight1, basis_coeff1):
    out1, fsum = pl.pallas_call(
        _rgcn_body,
        grid=(2, NBLK),
        in_specs=[
            pl.BlockSpec(memory_space=pltpu.SMEM),
            pl.BlockSpec(memory_space=pltpu.SMEM),
            pl.BlockSpec((REL, BLK, N), _adj_index),
            pl.BlockSpec((NB, N, H0), lambda p, i: (0, 0, 0)),
            pl.BlockSpec((NB, H0, H1), lambda p, i: (0, 0, 0)),
        ],
        out_specs=[
            pl.BlockSpec((BLK, H1),
                         lambda p, i: (jnp.where(p == 0, i, NBLK - 1 - i), 0)),
            pl.BlockSpec((2, H0), lambda p, i: (0, 0)),
        ],
        out_shape=[
            jax.ShapeDtypeStruct((N, H1), jnp.float32),
            jax.ShapeDtypeStruct((2, H0), jnp.float32),
        ],
        scratch_shapes=[
            pltpu.VMEM((N, H0), jnp.float32),
            pltpu.VMEM((REL, N, H0), jnp.bfloat16),
            pltpu.VMEM((REL, H0, H1), jnp.bfloat16),
            pltpu.VMEM((N, H0), jnp.bfloat16),
            pltpu.VMEM((REL, CACHE_BLKS * BLK, N), jnp.bfloat16),
        ],
        compiler_params=pltpu.CompilerParams(
            dimension_semantics=("arbitrary", "arbitrary"),
            vmem_limit_bytes=66584576),
    )(basis_coeff0, basis_coeff1, adj, basis_weight0, basis_weight1)
    final = fsum.reshape(1, H0 + H1)
    return (out1, final)


# speedup vs baseline: 1.3959x; 1.3959x over previous
import jax
import jax.numpy as jnp
from jax.experimental import pallas as pl
from jax.experimental.pallas import tpu as pltpu

N = 4096
REL = 2
NB = 2
H0 = 64
H1 = 64
BLK = 256
NBLK = N // BLK
CACHE_BLKS = 8


def _bf(x):
    return x.astype(jnp.bfloat16).astype(jnp.float32)


def _rgcn_body(c0_ref, c1_ref, adj_ref, bw0_ref, bw1_ref,
               out1_ref, fsum_ref, out0_scr, w0_scr, w1_scr, h_scr, cache_scr):
    p = pl.program_id(0)
    i = pl.program_id(1)

    @pl.when(jnp.logical_and(p == 0, i == 0))
    def _init():
        fsum_ref[...] = jnp.zeros_like(fsum_ref)
        for r in range(REL):
            w0_scr[r] = (_bf(c0_ref[r, 0]) * _bf(bw0_ref[0])
                         + _bf(c0_ref[r, 1]) * _bf(bw0_ref[1])
                         ).astype(jnp.bfloat16)
            w1_scr[r] = (_bf(c1_ref[r, 0]) * _bf(bw1_ref[0])
                         + _bf(c1_ref[r, 1]) * _bf(bw1_ref[1])
                         ).astype(jnp.bfloat16)

    @pl.when(p == 0)
    def _phase0():
        slot = jnp.minimum(i, CACHE_BLKS)
        acc = jnp.zeros((BLK, H0), dtype=jnp.float32)
        for r in range(REL):
            cache_scr[r, pl.ds(slot * BLK, BLK), :] = \
                adj_ref[r].astype(jnp.bfloat16)
            acc = acc + jnp.dot(cache_scr[r, pl.ds(slot * BLK, BLK), :],
                                w0_scr[r], preferred_element_type=jnp.float32)
        out0_scr[pl.ds(i * BLK, BLK), :] = acc
        fsum_ref[0:1, :] = fsum_ref[0:1, :] + jnp.sum(acc, axis=0, keepdims=True)

    @pl.when(jnp.logical_and(p == 1, i == 0))
    def _mk_h():
        h_scr[...] = jnp.maximum(out0_scr[...], 0.0).astype(jnp.bfloat16)

    @pl.when(p == 1)
    def _phase1():
        j = NBLK - 1 - i
        slot = jnp.minimum(j, CACHE_BLKS)

        @pl.when(j >= CACHE_BLKS)
        def _park_fresh():
            for r in range(REL):
                cache_scr[r, pl.ds(CACHE_BLKS * BLK, BLK), :] = \
                    adj_ref[r].astype(jnp.bfloat16)

        h = h_scr[...]
        acc = jnp.zeros((BLK, H1), dtype=jnp.float32)
        for r in range(REL):
            t = jnp.dot(cache_scr[r, pl.ds(slot * BLK, BLK), :], h,
                        preferred_element_type=jnp.float32)
            acc = acc + jnp.dot(t.astype(jnp.bfloat16), w1_scr[r],
                                preferred_element_type=jnp.float32)
        out1_ref[...] = acc
        fsum_ref[1:2, :] = fsum_ref[1:2, :] + jnp.sum(acc, axis=0, keepdims=True)


def _adj_index(p, i):
    return (0, jnp.where(p == 0, i, jnp.maximum(NBLK - 1 - i, CACHE_BLKS)), 0)


def kernel(adj, basis_weight0, basis_coeff0, basis_weight1, basis_coeff1):
    out1, fsum = pl.pallas_call(
        _rgcn_body,
        grid=(2, NBLK),
        in_specs=[
            pl.BlockSpec(memory_space=pltpu.SMEM),
            pl.BlockSpec(memory_space=pltpu.SMEM),
            pl.BlockSpec((REL, BLK, N), _adj_index),
            pl.BlockSpec((NB, N, H0), lambda p, i: (0, 0, 0)),
            pl.BlockSpec((NB, H0, H1), lambda p, i: (0, 0, 0)),
        ],
        out_specs=[
            pl.BlockSpec((BLK, H1),
                         lambda p, i: (jnp.where(p == 0, i, NBLK - 1 - i), 0)),
            pl.BlockSpec((2, H0), lambda p, i: (0, 0)),
        ],
        out_shape=[
            jax.ShapeDtypeStruct((N, H1), jnp.float32),
            jax.ShapeDtypeStruct((2, H0), jnp.float32),
        ],
        scratch_shapes=[
            pltpu.VMEM((N, H0), jnp.float32),
            pltpu.VMEM((REL, N, H0), jnp.bfloat16),
            pltpu.VMEM((REL, H0, H1), jnp.bfloat16),
            pltpu.VMEM((N, H0), jnp.bfloat16),
            pltpu.VMEM((REL, (CACHE_BLKS + 1) * BLK, N),
                       jnp.bfloat16),
        ],
        compiler_params=pltpu.CompilerParams(
            dimension_semantics=("arbitrary", "arbitrary"),
            vmem_limit_bytes=66584576),
    )(basis_coeff0, basis_coeff1, adj, basis_weight0, basis_weight1)
    final = fsum.reshape(1, H0 + H1)
    return (out1, final)
